# baseline (speedup 1.0000x reference)
import jax
import jax.numpy as jnp
from jax.experimental import pallas as pl

B = 4096
D_IN = 1024
D_HID = 1000
D_OUT = 1000
BLK = 512
NB = B // BLK


def _body(q_ref, x_ref,
          w10_ref, b10_ref, w20_ref, b20_ref,
          w11_ref, b11_ref, w21_ref, b21_ref,
          w12_ref, b12_ref, w22_ref, b22_ref,
          out_ref):
    xb = x_ref[...].astype(jnp.bfloat16)

    def head(w1_ref, b1_ref, w2_ref, b2_ref):
        h = jnp.tanh(
            jnp.dot(xb, w1_ref[...].astype(jnp.bfloat16),
                    preferred_element_type=jnp.float32)
            + b1_ref[...])
        return (jnp.dot(h.astype(jnp.bfloat16),
                        w2_ref[...].astype(jnp.bfloat16),
                        preferred_element_type=jnp.float32)
                + b2_ref[...])

    y0 = head(w10_ref, b10_ref, w20_ref, b20_ref)
    y1 = head(w11_ref, b11_ref, w21_ref, b21_ref)
    y2 = head(w12_ref, b12_ref, w22_ref, b22_ref)

    qb = q_ref[...]
    q0 = qb[:, 0:1]
    q1 = qb[:, 1:2]
    q2 = qb[:, 2:3]
    m0 = (q0 >= q1) & (q0 >= q2)
    m1 = jnp.logical_not(m0) & (q1 >= q2)
    out_ref[...] = jnp.where(m0, y0, jnp.where(m1, y1, y2))


def kernel(hidden_states, question_type_output,
           W1_yn, b1_yn, W2_yn, b2_yn,
           W1_num, b1_num, W2_num, b2_num,
           W1_oth, b1_oth, W2_oth, b2_oth):
    def pad_w2(w2):
        return jnp.pad(w2, ((0, 0), (0, D_OUT - w2.shape[1])))

    def pad_b2(b2):
        return jnp.pad(b2, ((0, D_OUT - b2.shape[0]),)).reshape(1, D_OUT)

    ws = [
        W1_yn, b1_yn.reshape(1, D_HID), pad_w2(W2_yn), pad_b2(b2_yn),
        W1_num, b1_num.reshape(1, D_HID), pad_w2(W2_num), pad_b2(b2_num),
        W1_oth, b1_oth.reshape(1, D_HID), pad_w2(W2_oth), pad_b2(b2_oth),
    ]

    full = lambda shape: pl.BlockSpec(shape, lambda i: (0,) * len(shape))
    w_specs = []
    for w in ws:
        w_specs.append(full(w.shape))

    return pl.pallas_call(
        _body,
        grid=(NB,),
        in_specs=[
            pl.BlockSpec((BLK, 3), lambda i: (i, 0)),
            pl.BlockSpec((BLK, D_IN), lambda i: (i, 0)),
            *w_specs,
        ],
        out_specs=pl.BlockSpec((BLK, D_OUT), lambda i: (i, 0)),
        out_shape=jax.ShapeDtypeStruct((B, D_OUT), jnp.float32),
    )(question_type_output, hidden_states, *ws)


# speedup vs baseline: 1.0014x; 1.0014x over previous
import jax
import jax.numpy as jnp
from jax.experimental import pallas as pl

B = 4096
D_IN = 1024
D_HID = 1000
D_OUT = 1000
BLK = 1024
NB = B // BLK


def _body(q_ref, x_ref,
          w10_ref, b10_ref, w20_ref, b20_ref,
          w11_ref, b11_ref, w21_ref, b21_ref,
          w12_ref, b12_ref, w22_ref, b22_ref,
          out_ref):
    xb = x_ref[...].astype(jnp.bfloat16)

    def head(w1_ref, b1_ref, w2_ref, b2_ref):
        h = jnp.tanh(
            jnp.dot(xb, w1_ref[...].astype(jnp.bfloat16),
                    preferred_element_type=jnp.float32)
            + b1_ref[...])
        return (jnp.dot(h.astype(jnp.bfloat16),
                        w2_ref[...].astype(jnp.bfloat16),
                        preferred_element_type=jnp.float32)
                + b2_ref[...])

    y0 = head(w10_ref, b10_ref, w20_ref, b20_ref)
    y1 = head(w11_ref, b11_ref, w21_ref, b21_ref)
    y2 = head(w12_ref, b12_ref, w22_ref, b22_ref)

    qb = q_ref[...]
    q0 = qb[:, 0:1]
    q1 = qb[:, 1:2]
    q2 = qb[:, 2:3]
    m0 = (q0 >= q1) & (q0 >= q2)
    m1 = jnp.logical_not(m0) & (q1 >= q2)
    out_ref[...] = jnp.where(m0, y0, jnp.where(m1, y1, y2))


def kernel(hidden_states, question_type_output,
           W1_yn, b1_yn, W2_yn, b2_yn,
           W1_num, b1_num, W2_num, b2_num,
           W1_oth, b1_oth, W2_oth, b2_oth):
    def pad_w2(w2):
        return jnp.pad(w2, ((0, 0), (0, D_OUT - w2.shape[1])))

    def pad_b2(b2):
        return jnp.pad(b2, ((0, D_OUT - b2.shape[0]),)).reshape(1, D_OUT)

    ws = [
        W1_yn, b1_yn.reshape(1, D_HID), pad_w2(W2_yn), pad_b2(b2_yn),
        W1_num, b1_num.reshape(1, D_HID), pad_w2(W2_num), pad_b2(b2_num),
        W1_oth, b1_oth.reshape(1, D_HID), pad_w2(W2_oth), pad_b2(b2_oth),
    ]

    full = lambda shape: pl.BlockSpec(shape, lambda i: (0,) * len(shape))
    w_specs = []
    for w in ws:
        w_specs.append(full(w.shape))

    return pl.pallas_call(
        _body,
        grid=(NB,),
        in_specs=[
            pl.BlockSpec((BLK, 3), lambda i: (i, 0)),
            pl.BlockSpec((BLK, D_IN), lambda i: (i, 0)),
            *w_specs,
        ],
        out_specs=pl.BlockSpec((BLK, D_OUT), lambda i: (i, 0)),
        out_shape=jax.ShapeDtypeStruct((B, D_OUT), jnp.float32),
    )(question_type_output, hidden_states, *ws)
